# Initial kernel scaffold; baseline (speedup 1.0000x reference)
import jax
import jax.numpy as jnp
from jax.experimental import pallas as pl


def kernel(hidden_states, cos, sin, wq_a, q_a_norm_w, wq_b, wkv_a, kv_a_norm_w, wkv_b, wo, idx_wq_b, idx_wk, idx_k_norm_w, idx_w_proj):
    raise NotImplementedError("write your pallas kernel here")


# speedup vs baseline: 14.4447x; 14.4447x over previous
import functools

import jax
import jax.numpy as jnp
from jax.experimental import pallas as pl
from jax.experimental.pallas import tpu as pltpu

S = 2048
HID = 2048
H = 16
Q_LORA, KV_LORA = 1024, 512
ROPE_D, NOPE_D, V_D = 64, 128, 128
QK_D = ROPE_D + NOPE_D
IDX_K, IDX_H, IDX_D = 1024, 16, 128
EPS = 1e-5
NEG = -1e9

BM = 256
BF = jnp.bfloat16
HIP = jax.lax.Precision.HIGHEST


def _rope(x, cos, sin):
    hd = ROPE_D // 2
    rot = jnp.concatenate([-x[:, hd:], x[:, :hd]], axis=1)
    return x * cos + rot * sin


def _rms(x, w):
    var = jnp.mean(jnp.square(x), axis=-1, keepdims=True)
    return x * jax.lax.rsqrt(var + EPS) * w


def _stage1_kernel(h_ref, w_ref, cosb_ref, sinb_ref, qnw_ref, kvnw_ref, iknw_ref,
                   qr_ref, kvc_ref, krope_ref, ik_ref, wh_ref):
    h = h_ref[...]
    acc = jnp.dot(h, w_ref[...], preferred_element_type=jnp.float32)
    cosb = cosb_ref[...]
    sinb = sinb_ref[...]
    ik_off = Q_LORA + KV_LORA + ROPE_D + 64
    wh_off = ik_off + IDX_D
    qr_ref[...] = _rms(acc[:, :Q_LORA], qnw_ref[...])
    kvc_ref[...] = _rms(acc[:, Q_LORA:Q_LORA + KV_LORA], kvnw_ref[...])
    krope_ref[...] = _rope(acc[:, Q_LORA + KV_LORA:Q_LORA + KV_LORA + ROPE_D], cosb, sinb)
    ikn = _rms(acc[:, ik_off:ik_off + IDX_D], iknw_ref[...])
    ik_ref[...] = jnp.concatenate(
        [_rope(ikn[:, :ROPE_D], cosb, sinb), ikn[:, ROPE_D:]], axis=1)
    wh_ref[...] = acc[:, wh_off:wh_off + IDX_H] * (IDX_H ** -0.5)


def _stage2_kernel(qr_ref, wqb_ref, kvc_ref, wkvb_ref, krope_ref, cosb_ref, sinb_ref,
                   q_ref, k_ref, v_ref):
    qh = jnp.dot(qr_ref[...].astype(BF), wqb_ref[0],
                 preferred_element_type=jnp.float32)
    cosb = cosb_ref[...]
    sinb = sinb_ref[...]
    qrope = _rope(qh[:, NOPE_D:], cosb, sinb)
    q_ref[0] = jnp.concatenate([qh[:, :NOPE_D], qrope], axis=1).astype(BF)
    kvh = jnp.dot(kvc_ref[...].astype(BF), wkvb_ref[0],
                  preferred_element_type=jnp.float32)
    k_ref[0] = jnp.concatenate(
        [kvh[:, :NOPE_D], krope_ref[...]], axis=1).astype(BF)
    v_ref[0] = kvh[:, NOPE_D:].astype(BF)


def _stage2b_kernel(qr_ref, wiqb_ref, cosb_ref, sinb_ref, iq_ref):
    iqh = jnp.dot(qr_ref[...].astype(BF), wiqb_ref[0],
                  preferred_element_type=jnp.float32)
    iq_ref[0] = jnp.concatenate(
        [_rope(iqh[:, :ROPE_D], cosb_ref[...], sinb_ref[...]), iqh[:, ROPE_D:]],
        axis=1).astype(BF)


def _stage3a_kernel(iq_ref, ik_ref, wh_ref, mask_ref):
    m = pl.program_id(0)
    bm, seq = mask_ref.shape
    rows = m * bm + jax.lax.broadcasted_iota(jnp.int32, (bm, seq), 0)
    cols = jax.lax.broadcasted_iota(jnp.int32, (bm, seq), 1)
    causal = cols <= rows
    mask_ref[...] = jnp.where(causal, 0.0, NEG)

    @pl.when((m + 1) * bm > IDX_K)
    def _():
        ik = ik_ref[...].astype(BF)
        wh = wh_ref[...].astype(BF).astype(jnp.float32)
        isc = jnp.zeros((bm, seq), jnp.float32)
        for n in range(IDX_H):
            sn = jax.lax.dot_general(
                iq_ref[n], ik, (((1,), (1,)), ((), ())),
                preferred_element_type=jnp.float32)
            rn = (jax.nn.relu(sn) * (IDX_D ** -0.5)).astype(BF).astype(jnp.float32)
            isc = isc + rn * wh[:, n:n + 1]
        isc = jnp.where(causal, isc, NEG)
        lo = jnp.min(jnp.where(causal, isc, -NEG), axis=1, keepdims=True)
        hi = jnp.max(isc, axis=1, keepdims=True)

        def body(_, lohi):
            lo, hi = lohi
            mid = 0.5 * (lo + hi)
            cnt = jnp.sum((isc >= mid).astype(jnp.float32), axis=1, keepdims=True)
            pred = cnt >= IDX_K
            return jnp.where(pred, mid, lo), jnp.where(pred, hi, mid)

        lo, hi = jax.lax.fori_loop(0, 42, body, (lo, hi))
        mask_ref[...] = jnp.where(isc >= lo, 0.0, NEG)


def _stage3b_kernel(q_ref, k_ref, v_ref, mask_ref, out_ref):
    scale = QK_D ** -0.5
    logits = jax.lax.dot_general(
        q_ref[0], k_ref[0], (((1,), (1,)), ((), ())),
        preferred_element_type=jnp.float32) * scale + mask_ref[...]
    mx = jnp.max(logits, axis=1, keepdims=True)
    p = jnp.exp(logits - mx)
    denom = jnp.sum(p, axis=1, keepdims=True)
    o = jnp.dot(p.astype(BF), v_ref[0], preferred_element_type=jnp.float32)
    out_ref[0] = (o / denom).astype(BF)


def _stage4_kernel(x_ref, w_ref, o_ref):
    acc = jnp.zeros(o_ref.shape, jnp.float32)
    for h in range(H):
        acc = acc + jnp.dot(x_ref[h], w_ref[h], preferred_element_type=jnp.float32)
    o_ref[...] = acc


def kernel(hidden_states, cos, sin, wq_a, q_a_norm_w, wq_b, wkv_a, kv_a_norm_w,
           wkv_b, wo, idx_wq_b, idx_wk, idx_k_norm_w, idx_w_proj):
    b, s, hid = hidden_states.shape
    h2d = hidden_states.reshape(s, hid)
    cosb = cos.reshape(s, ROPE_D)
    sinb = sin.reshape(s, ROPE_D)
    nm = s // BM

    n1 = Q_LORA + KV_LORA + ROPE_D + 64 + IDX_D + 128
    rhs1 = jnp.concatenate([
        wq_a, wkv_a, jnp.zeros((hid, 64), jnp.float32), idx_wk, idx_w_proj,
        jnp.zeros((hid, 128 - IDX_H), jnp.float32)], axis=1).astype(BF)
    h2d = h2d.astype(BF)
    qr, kvc, krope, ik, wh = pl.pallas_call(
        _stage1_kernel,
        grid=(nm,),
        in_specs=[
            pl.BlockSpec((BM, hid), lambda i: (i, 0)),
            pl.BlockSpec((hid, n1), lambda i: (0, 0)),
            pl.BlockSpec((BM, ROPE_D), lambda i: (i, 0)),
            pl.BlockSpec((BM, ROPE_D), lambda i: (i, 0)),
            pl.BlockSpec((1, Q_LORA), lambda i: (0, 0)),
            pl.BlockSpec((1, KV_LORA), lambda i: (0, 0)),
            pl.BlockSpec((1, IDX_D), lambda i: (0, 0)),
        ],
        out_specs=[
            pl.BlockSpec((BM, Q_LORA), lambda i: (i, 0)),
            pl.BlockSpec((BM, KV_LORA), lambda i: (i, 0)),
            pl.BlockSpec((BM, ROPE_D), lambda i: (i, 0)),
            pl.BlockSpec((BM, IDX_D), lambda i: (i, 0)),
            pl.BlockSpec((BM, IDX_H), lambda i: (i, 0)),
        ],
        out_shape=[
            jax.ShapeDtypeStruct((s, Q_LORA), jnp.float32),
            jax.ShapeDtypeStruct((s, KV_LORA), jnp.float32),
            jax.ShapeDtypeStruct((s, ROPE_D), jnp.float32),
            jax.ShapeDtypeStruct((s, IDX_D), jnp.float32),
            jax.ShapeDtypeStruct((s, IDX_H), jnp.float32),
        ],
    )(h2d, rhs1, cosb, sinb, q_a_norm_w.reshape(1, -1),
      kv_a_norm_w.reshape(1, -1), idx_k_norm_w.reshape(1, -1))

    wqb3 = wq_b.reshape(Q_LORA, H, QK_D).transpose(1, 0, 2).astype(BF)
    wkvb3 = wkv_b.reshape(KV_LORA, H, NOPE_D + V_D).transpose(1, 0, 2).astype(BF)
    qf, k, v = pl.pallas_call(
        _stage2_kernel,
        grid=(nm, H),
        in_specs=[
            pl.BlockSpec((BM, Q_LORA), lambda i, j: (i, 0)),
            pl.BlockSpec((1, Q_LORA, QK_D), lambda i, j: (j, 0, 0)),
            pl.BlockSpec((BM, KV_LORA), lambda i, j: (i, 0)),
            pl.BlockSpec((1, KV_LORA, NOPE_D + V_D), lambda i, j: (j, 0, 0)),
            pl.BlockSpec((BM, ROPE_D), lambda i, j: (i, 0)),
            pl.BlockSpec((BM, ROPE_D), lambda i, j: (i, 0)),
            pl.BlockSpec((BM, ROPE_D), lambda i, j: (i, 0)),
        ],
        out_specs=[
            pl.BlockSpec((1, BM, QK_D), lambda i, j: (j, i, 0)),
            pl.BlockSpec((1, BM, QK_D), lambda i, j: (j, i, 0)),
            pl.BlockSpec((1, BM, V_D), lambda i, j: (j, i, 0)),
        ],
        out_shape=[
            jax.ShapeDtypeStruct((H, s, QK_D), BF),
            jax.ShapeDtypeStruct((H, s, QK_D), BF),
            jax.ShapeDtypeStruct((H, s, V_D), BF),
        ],
    )(qr, wqb3, kvc, wkvb3, krope, cosb, sinb)

    s2 = s - IDX_K
    nm2 = s2 // BM
    kb = IDX_K // BM
    wiqb3 = idx_wq_b.reshape(Q_LORA, IDX_H, IDX_D).transpose(1, 0, 2).astype(BF)
    iq = pl.pallas_call(
        _stage2b_kernel,
        grid=(nm2, IDX_H),
        in_specs=[
            pl.BlockSpec((BM, Q_LORA), lambda i, j: (i + kb, 0)),
            pl.BlockSpec((1, Q_LORA, IDX_D), lambda i, j: (j, 0, 0)),
            pl.BlockSpec((BM, ROPE_D), lambda i, j: (i + kb, 0)),
            pl.BlockSpec((BM, ROPE_D), lambda i, j: (i + kb, 0)),
        ],
        out_specs=pl.BlockSpec((1, BM, IDX_D), lambda i, j: (j, i, 0)),
        out_shape=jax.ShapeDtypeStruct((IDX_H, s2, IDX_D), BF),
    )(qr, wiqb3, cosb, sinb)

    mask = pl.pallas_call(
        _stage3a_kernel,
        grid=(nm,),
        in_specs=[
            pl.BlockSpec((IDX_H, BM, IDX_D), lambda i: (0, jnp.maximum(i - kb, 0), 0)),
            pl.BlockSpec((s, IDX_D), lambda i: (0, 0)),
            pl.BlockSpec((BM, IDX_H), lambda i: (i, 0)),
        ],
        out_specs=pl.BlockSpec((BM, s), lambda i: (i, 0)),
        out_shape=jax.ShapeDtypeStruct((s, s), jnp.float32),
    )(iq, ik, wh)

    attn = pl.pallas_call(
        _stage3b_kernel,
        grid=(nm, H),
        in_specs=[
            pl.BlockSpec((1, BM, QK_D), lambda i, j: (j, i, 0)),
            pl.BlockSpec((1, s, QK_D), lambda i, j: (j, 0, 0)),
            pl.BlockSpec((1, s, V_D), lambda i, j: (j, 0, 0)),
            pl.BlockSpec((BM, s), lambda i, j: (i, 0)),
        ],
        out_specs=pl.BlockSpec((1, BM, V_D), lambda i, j: (j, i, 0)),
        out_shape=jax.ShapeDtypeStruct((H, s, V_D), BF),
    )(qf, k, v, mask)

    wo3 = wo.reshape(H, V_D, hid).astype(BF)
    out = pl.pallas_call(
        _stage4_kernel,
        grid=(nm,),
        in_specs=[
            pl.BlockSpec((H, BM, V_D), lambda i: (0, i, 0)),
            pl.BlockSpec((H, V_D, hid), lambda i: (0, 0, 0)),
        ],
        out_specs=pl.BlockSpec((BM, hid), lambda i: (i, 0)),
        out_shape=jax.ShapeDtypeStruct((s, hid), jnp.float32),
    )(attn, wo3)

    return out.reshape(b, s, hid)


# speedup vs baseline: 15.0228x; 1.0400x over previous
import functools

import jax
import jax.numpy as jnp
from jax.experimental import pallas as pl
from jax.experimental.pallas import tpu as pltpu

S = 2048
HID = 2048
H = 16
Q_LORA, KV_LORA = 1024, 512
ROPE_D, NOPE_D, V_D = 64, 128, 128
QK_D = ROPE_D + NOPE_D
IDX_K, IDX_H, IDX_D = 1024, 16, 128
EPS = 1e-5
NEG = -1e9

BM = 256
BF = jnp.bfloat16
HIP = jax.lax.Precision.HIGHEST


def _rope(x, cos, sin):
    hd = ROPE_D // 2
    rot = jnp.concatenate([-x[:, hd:], x[:, :hd]], axis=1)
    return x * cos + rot * sin


def _rms(x, w):
    var = jnp.mean(jnp.square(x), axis=-1, keepdims=True)
    return x * jax.lax.rsqrt(var + EPS) * w


def _stage1_kernel(h_ref, w_ref, cosb_ref, sinb_ref, qnw_ref, kvnw_ref, iknw_ref,
                   qr_ref, kvc_ref, krope_ref, ik_ref, wh_ref):
    h = h_ref[...]
    acc = jnp.dot(h, w_ref[...], preferred_element_type=jnp.float32)
    cosb = cosb_ref[...]
    sinb = sinb_ref[...]
    ik_off = Q_LORA + KV_LORA + ROPE_D + 64
    wh_off = ik_off + IDX_D
    qr_ref[...] = _rms(acc[:, :Q_LORA], qnw_ref[...])
    kvc_ref[...] = _rms(acc[:, Q_LORA:Q_LORA + KV_LORA], kvnw_ref[...])
    krope_ref[...] = _rope(acc[:, Q_LORA + KV_LORA:Q_LORA + KV_LORA + ROPE_D], cosb, sinb)
    ikn = _rms(acc[:, ik_off:ik_off + IDX_D], iknw_ref[...])
    ik_ref[...] = jnp.concatenate(
        [_rope(ikn[:, :ROPE_D], cosb, sinb), ikn[:, ROPE_D:]], axis=1)
    wh_ref[...] = acc[:, wh_off:wh_off + IDX_H] * (IDX_H ** -0.5)


def _stage2_kernel(qr_ref, wqb_ref, kvc_ref, wkvb_ref, krope_ref, cosb_ref, sinb_ref,
                   q_ref, k_ref, v_ref):
    qh = jnp.dot(qr_ref[...].astype(BF), wqb_ref[0],
                 preferred_element_type=jnp.float32)
    cosb = cosb_ref[...]
    sinb = sinb_ref[...]
    qrope = _rope(qh[:, NOPE_D:], cosb, sinb)
    q_ref[0] = jnp.concatenate([qh[:, :NOPE_D], qrope], axis=1).astype(BF)
    kvh = jnp.dot(kvc_ref[...].astype(BF), wkvb_ref[0],
                  preferred_element_type=jnp.float32)
    k_ref[0] = jnp.concatenate(
        [kvh[:, :NOPE_D], krope_ref[...]], axis=1).astype(BF)
    v_ref[0] = kvh[:, NOPE_D:].astype(BF)


def _stage2b_kernel(qr_ref, wiqb_ref, cosb_ref, sinb_ref, iq_ref):
    iqh = jnp.dot(qr_ref[...].astype(BF), wiqb_ref[0],
                  preferred_element_type=jnp.float32)
    iq_ref[0] = jnp.concatenate(
        [_rope(iqh[:, :ROPE_D], cosb_ref[...], sinb_ref[...]), iqh[:, ROPE_D:]],
        axis=1).astype(BF)


def _stage3a_kernel(iq_ref, ik_ref, wh_ref, mask_ref):
    m = pl.program_id(0)
    bm, seq = mask_ref.shape
    rows = IDX_K + m * bm + jax.lax.broadcasted_iota(jnp.int32, (bm, seq), 0)
    cols = jax.lax.broadcasted_iota(jnp.int32, (bm, seq), 1)
    causal = cols <= rows
    if True:
        ik = ik_ref[...].astype(BF)
        wh = wh_ref[...].astype(BF).astype(jnp.float32)
        isc = jnp.zeros((bm, seq), jnp.float32)
        for n in range(IDX_H):
            sn = jax.lax.dot_general(
                iq_ref[n], ik, (((1,), (1,)), ((), ())),
                preferred_element_type=jnp.float32)
            rn = (jax.nn.relu(sn) * (IDX_D ** -0.5)).astype(BF).astype(jnp.float32)
            isc = isc + rn * wh[:, n:n + 1]
        isc = jnp.where(causal, isc, NEG)
        lo = jnp.min(jnp.where(causal, isc, -NEG), axis=1, keepdims=True)
        hi = jnp.max(isc, axis=1, keepdims=True)

        def body(_, lohi):
            lo, hi = lohi
            mid = 0.5 * (lo + hi)
            cnt = jnp.sum((isc >= mid).astype(jnp.float32), axis=1, keepdims=True)
            pred = cnt >= IDX_K
            return jnp.where(pred, mid, lo), jnp.where(pred, hi, mid)

        lo, hi = jax.lax.fori_loop(0, 42, body, (lo, hi))
        mask_ref[...] = jnp.where(isc >= lo, 0.0, NEG)


def _stage3b_first_kernel(q_ref, k_ref, v_ref, out_ref):
    m = pl.program_id(0)
    bm = q_ref.shape[1]
    seq = k_ref.shape[1]
    rows = m * bm + jax.lax.broadcasted_iota(jnp.int32, (bm, seq), 0)
    cols = jax.lax.broadcasted_iota(jnp.int32, (bm, seq), 1)
    mask = jnp.where(cols <= rows, 0.0, NEG)
    scale = QK_D ** -0.5
    logits = jax.lax.dot_general(
        q_ref[0], k_ref[0], (((1,), (1,)), ((), ())),
        preferred_element_type=jnp.float32) * scale + mask
    mx = jnp.max(logits, axis=1, keepdims=True)
    p = jnp.exp(logits - mx)
    denom = jnp.sum(p, axis=1, keepdims=True)
    o = jnp.dot(p.astype(BF), v_ref[0], preferred_element_type=jnp.float32)
    out_ref[0] = (o / denom).astype(BF)


def _stage3b_kernel(q_ref, k_ref, v_ref, mask_ref, out_ref):
    scale = QK_D ** -0.5
    logits = jax.lax.dot_general(
        q_ref[0], k_ref[0], (((1,), (1,)), ((), ())),
        preferred_element_type=jnp.float32) * scale + mask_ref[...]
    mx = jnp.max(logits, axis=1, keepdims=True)
    p = jnp.exp(logits - mx)
    denom = jnp.sum(p, axis=1, keepdims=True)
    o = jnp.dot(p.astype(BF), v_ref[0], preferred_element_type=jnp.float32)
    out_ref[0] = (o / denom).astype(BF)


def _stage4_kernel(x_ref, w_ref, o_ref):
    acc = jnp.zeros(o_ref.shape, jnp.float32)
    for h in range(H):
        acc = acc + jnp.dot(x_ref[h], w_ref[h], preferred_element_type=jnp.float32)
    o_ref[...] = acc


def kernel(hidden_states, cos, sin, wq_a, q_a_norm_w, wq_b, wkv_a, kv_a_norm_w,
           wkv_b, wo, idx_wq_b, idx_wk, idx_k_norm_w, idx_w_proj):
    b, s, hid = hidden_states.shape
    h2d = hidden_states.reshape(s, hid)
    cosb = cos.reshape(s, ROPE_D)
    sinb = sin.reshape(s, ROPE_D)
    nm = s // BM

    n1 = Q_LORA + KV_LORA + ROPE_D + 64 + IDX_D + 128
    rhs1 = jnp.concatenate([
        wq_a, wkv_a, jnp.zeros((hid, 64), jnp.float32), idx_wk, idx_w_proj,
        jnp.zeros((hid, 128 - IDX_H), jnp.float32)], axis=1).astype(BF)
    h2d = h2d.astype(BF)
    qr, kvc, krope, ik, wh = pl.pallas_call(
        _stage1_kernel,
        grid=(nm,),
        in_specs=[
            pl.BlockSpec((BM, hid), lambda i: (i, 0)),
            pl.BlockSpec((hid, n1), lambda i: (0, 0)),
            pl.BlockSpec((BM, ROPE_D), lambda i: (i, 0)),
            pl.BlockSpec((BM, ROPE_D), lambda i: (i, 0)),
            pl.BlockSpec((1, Q_LORA), lambda i: (0, 0)),
            pl.BlockSpec((1, KV_LORA), lambda i: (0, 0)),
            pl.BlockSpec((1, IDX_D), lambda i: (0, 0)),
        ],
        out_specs=[
            pl.BlockSpec((BM, Q_LORA), lambda i: (i, 0)),
            pl.BlockSpec((BM, KV_LORA), lambda i: (i, 0)),
            pl.BlockSpec((BM, ROPE_D), lambda i: (i, 0)),
            pl.BlockSpec((BM, IDX_D), lambda i: (i, 0)),
            pl.BlockSpec((BM, IDX_H), lambda i: (i, 0)),
        ],
        out_shape=[
            jax.ShapeDtypeStruct((s, Q_LORA), jnp.float32),
            jax.ShapeDtypeStruct((s, KV_LORA), jnp.float32),
            jax.ShapeDtypeStruct((s, ROPE_D), jnp.float32),
            jax.ShapeDtypeStruct((s, IDX_D), jnp.float32),
            jax.ShapeDtypeStruct((s, IDX_H), jnp.float32),
        ],
    )(h2d, rhs1, cosb, sinb, q_a_norm_w.reshape(1, -1),
      kv_a_norm_w.reshape(1, -1), idx_k_norm_w.reshape(1, -1))

    wqb3 = wq_b.reshape(Q_LORA, H, QK_D).transpose(1, 0, 2).astype(BF)
    wkvb3 = wkv_b.reshape(KV_LORA, H, NOPE_D + V_D).transpose(1, 0, 2).astype(BF)
    qf, k, v = pl.pallas_call(
        _stage2_kernel,
        grid=(nm, H),
        in_specs=[
            pl.BlockSpec((BM, Q_LORA), lambda i, j: (i, 0)),
            pl.BlockSpec((1, Q_LORA, QK_D), lambda i, j: (j, 0, 0)),
            pl.BlockSpec((BM, KV_LORA), lambda i, j: (i, 0)),
            pl.BlockSpec((1, KV_LORA, NOPE_D + V_D), lambda i, j: (j, 0, 0)),
            pl.BlockSpec((BM, ROPE_D), lambda i, j: (i, 0)),
            pl.BlockSpec((BM, ROPE_D), lambda i, j: (i, 0)),
            pl.BlockSpec((BM, ROPE_D), lambda i, j: (i, 0)),
        ],
        out_specs=[
            pl.BlockSpec((1, BM, QK_D), lambda i, j: (j, i, 0)),
            pl.BlockSpec((1, BM, QK_D), lambda i, j: (j, i, 0)),
            pl.BlockSpec((1, BM, V_D), lambda i, j: (j, i, 0)),
        ],
        out_shape=[
            jax.ShapeDtypeStruct((H, s, QK_D), BF),
            jax.ShapeDtypeStruct((H, s, QK_D), BF),
            jax.ShapeDtypeStruct((H, s, V_D), BF),
        ],
    )(qr, wqb3, kvc, wkvb3, krope, cosb, sinb)

    s2 = s - IDX_K
    nm2 = s2 // BM
    kb = IDX_K // BM
    wiqb3 = idx_wq_b.reshape(Q_LORA, IDX_H, IDX_D).transpose(1, 0, 2).astype(BF)
    iq = pl.pallas_call(
        _stage2b_kernel,
        grid=(nm2, IDX_H),
        in_specs=[
            pl.BlockSpec((BM, Q_LORA), lambda i, j: (i + kb, 0)),
            pl.BlockSpec((1, Q_LORA, IDX_D), lambda i, j: (j, 0, 0)),
            pl.BlockSpec((BM, ROPE_D), lambda i, j: (i + kb, 0)),
            pl.BlockSpec((BM, ROPE_D), lambda i, j: (i + kb, 0)),
        ],
        out_specs=pl.BlockSpec((1, BM, IDX_D), lambda i, j: (j, i, 0)),
        out_shape=jax.ShapeDtypeStruct((IDX_H, s2, IDX_D), BF),
    )(qr, wiqb3, cosb, sinb)

    mask = pl.pallas_call(
        _stage3a_kernel,
        grid=(nm2,),
        in_specs=[
            pl.BlockSpec((IDX_H, BM, IDX_D), lambda i: (0, i, 0)),
            pl.BlockSpec((s, IDX_D), lambda i: (0, 0)),
            pl.BlockSpec((BM, IDX_H), lambda i: (i + kb, 0)),
        ],
        out_specs=pl.BlockSpec((BM, s), lambda i: (i, 0)),
        out_shape=jax.ShapeDtypeStruct((s2, s), jnp.float32),
    )(iq, ik, wh)

    attn1 = pl.pallas_call(
        _stage3b_first_kernel,
        grid=(kb, H),
        in_specs=[
            pl.BlockSpec((1, BM, QK_D), lambda i, j: (j, i, 0)),
            pl.BlockSpec((1, IDX_K, QK_D), lambda i, j: (j, 0, 0)),
            pl.BlockSpec((1, IDX_K, V_D), lambda i, j: (j, 0, 0)),
        ],
        out_specs=pl.BlockSpec((1, BM, V_D), lambda i, j: (j, i, 0)),
        out_shape=jax.ShapeDtypeStruct((H, IDX_K, V_D), BF),
    )(qf, k, v)
    attn2 = pl.pallas_call(
        _stage3b_kernel,
        grid=(nm2, H),
        in_specs=[
            pl.BlockSpec((1, BM, QK_D), lambda i, j: (j, i + kb, 0)),
            pl.BlockSpec((1, s, QK_D), lambda i, j: (j, 0, 0)),
            pl.BlockSpec((1, s, V_D), lambda i, j: (j, 0, 0)),
            pl.BlockSpec((BM, s), lambda i, j: (i, 0)),
        ],
        out_specs=pl.BlockSpec((1, BM, V_D), lambda i, j: (j, i, 0)),
        out_shape=jax.ShapeDtypeStruct((H, s2, V_D), BF),
    )(qf, k, v, mask)
    attn = jnp.concatenate([attn1, attn2], axis=1)

    wo3 = wo.reshape(H, V_D, hid).astype(BF)
    out = pl.pallas_call(
        _stage4_kernel,
        grid=(nm,),
        in_specs=[
            pl.BlockSpec((H, BM, V_D), lambda i: (0, i, 0)),
            pl.BlockSpec((H, V_D, hid), lambda i: (0, 0, 0)),
        ],
        out_specs=pl.BlockSpec((BM, hid), lambda i: (i, 0)),
        out_shape=jax.ShapeDtypeStruct((s, hid), jnp.float32),
    )(attn, wo3)

    return out.reshape(b, s, hid)
